# baseline (speedup 1.0000x reference)
import functools

import jax
import jax.numpy as jnp
from jax import lax
from jax.experimental import pallas as pl
from jax.experimental.pallas import tpu as pltpu
from jax.experimental.pallas import tpu_sc as plsc

B = 4096
S = 50
SP = 56
D = 64
G = 2
CI = G * SP


@functools.cache
def _build_lookup():
    info = plsc.get_sparse_core_info()
    nc, ns = info.num_cores, info.num_subcores
    nw = nc * ns
    rows_per_w = B // nw
    nchunk = rows_per_w // G
    nbuf = 8
    lead = 4

    mesh = plsc.VectorSubcoreMesh(core_axis_name="c", subcore_axis_name="s")

    def body(idx_hbm, table_hbm, out_hbm, idx_v, rows_v, gsem, ssem):
        wid = lax.axis_index("s") * nc + lax.axis_index("c")
        row0 = wid * rows_per_w
        pltpu.sync_copy(idx_hbm.at[wid], idx_v)

        def gather(j, b):
            pltpu.async_copy(
                table_hbm.at[idx_v.at[j]], rows_v.at[b], gsem.at[b])

        def gather_wait(j, b):
            pltpu.make_async_copy(
                table_hbm.at[idx_v.at[j]], rows_v.at[b], gsem.at[b]).wait()

        def scatter(j, b):
            pltpu.async_copy(
                rows_v.at[b, pl.ds(0, SP)], out_hbm.at[row0 + G * j], ssem.at[b])
            pltpu.async_copy(
                rows_v.at[b, pl.ds(SP, SP)], out_hbm.at[row0 + G * j + 1],
                ssem.at[b])

        def scatter_wait(j, b):
            pltpu.make_async_copy(
                rows_v.at[b, pl.ds(0, SP)], out_hbm.at[row0 + G * j],
                ssem.at[b]).wait()
            pltpu.make_async_copy(
                rows_v.at[b, pl.ds(SP, SP)], out_hbm.at[row0 + G * j + 1],
                ssem.at[b]).wait()

        for p in range(lead):
            gather(p, p)

        @pl.loop(0, nchunk)
        def _(j):
            b = lax.rem(j, nbuf)
            jn = j + lead

            @pl.when(jn < nchunk)
            def _():
                bn = lax.rem(jn, nbuf)

                @pl.when(jn >= nbuf)
                def _():
                    scatter_wait(jn - nbuf, bn)

                gather(jn, bn)

            gather_wait(j, b)
            scatter(j, b)

        for t in range(nbuf):
            j = nchunk - nbuf + t
            scatter_wait(j, j % nbuf)

    return pl.kernel(
        body,
        out_type=jax.ShapeDtypeStruct((B, SP, D), jnp.float32),
        mesh=mesh,
        scratch_types=[
            pltpu.VMEM((nchunk, CI), jnp.int32),
            pltpu.VMEM((nbuf, CI, D), jnp.float32),
            pltpu.SemaphoreType.DMA((nbuf,)),
            pltpu.SemaphoreType.DMA((nbuf,)),
        ],
        compiler_params=pltpu.CompilerParams(use_tc_tiling_on_sc=False),
    ), nw, nchunk


def kernel(token_ids, emb):
    lookup, nw, nchunk = _build_lookup()
    ids = token_ids.reshape(nw, nchunk * G, S)
    ids = jnp.pad(ids, ((0, 0), (0, 0), (0, SP - S)))
    ids = ids.reshape(nw, nchunk, CI)
    return lookup(ids, emb)[:, :S, :]


# speedup vs baseline: 3.2772x; 3.2772x over previous
import functools

import jax
import jax.numpy as jnp
from jax import lax
from jax.experimental import pallas as pl
from jax.experimental.pallas import tpu as pltpu
from jax.experimental.pallas import tpu_sc as plsc

B = 4096
S = 50
D = 64
G = 8


@functools.cache
def _build_lookup():
    info = plsc.get_sparse_core_info()
    nc, ns = info.num_cores, info.num_subcores
    nw = nc * ns
    rows_per_w = B // nw
    nchunk = rows_per_w // G
    nbuf = 4
    lead = 2

    mesh = plsc.VectorSubcoreMesh(core_axis_name="c", subcore_axis_name="s")

    def body(idx_hbm, table_hbm, out_hbm, idx_v, rows_v, gsem, ssem):
        wid = lax.axis_index("s") * nc + lax.axis_index("c")
        row0 = wid * rows_per_w
        pltpu.sync_copy(idx_hbm.at[wid], idx_v)

        def gather(j, b):
            for k in range(G):
                pltpu.async_copy(
                    table_hbm.at[idx_v.at[G * j + k]], rows_v.at[b, k],
                    gsem.at[b])

        def gather_wait(j, b):
            for k in range(G):
                pltpu.make_async_copy(
                    table_hbm.at[idx_v.at[G * j + k]], rows_v.at[b, k],
                    gsem.at[b]).wait()

        def scatter(j, b):
            pltpu.async_copy(
                rows_v.at[b], out_hbm.at[pl.ds(row0 + G * j, G)], ssem.at[b])

        def scatter_wait(j, b):
            pltpu.make_async_copy(
                rows_v.at[b], out_hbm.at[pl.ds(row0 + G * j, G)],
                ssem.at[b]).wait()

        for p in range(lead):
            gather(p, p)

        @pl.loop(0, nchunk)
        def _(j):
            b = lax.rem(j, nbuf)
            jn = j + lead

            @pl.when(jn < nchunk)
            def _():
                bn = lax.rem(jn, nbuf)

                @pl.when(jn >= nbuf)
                def _():
                    scatter_wait(jn - nbuf, bn)

                gather(jn, bn)

            gather_wait(j, b)
            scatter(j, b)

        for t in range(nbuf):
            j = nchunk - nbuf + t
            scatter_wait(j, j % nbuf)

    return pl.kernel(
        body,
        out_type=jax.ShapeDtypeStruct((B, S, D), jnp.float32),
        mesh=mesh,
        scratch_types=[
            pltpu.VMEM((rows_per_w, S), jnp.int32),
            pltpu.VMEM((nbuf, G, S, D), jnp.float32),
            pltpu.SemaphoreType.DMA((nbuf,)),
            pltpu.SemaphoreType.DMA((nbuf,)),
        ],
        compiler_params=pltpu.CompilerParams(use_tc_tiling_on_sc=False),
    ), nw, rows_per_w


def kernel(token_ids, emb):
    lookup, nw, rows_per_w = _build_lookup()
    ids = token_ids.reshape(nw, rows_per_w, S)
    return lookup(ids, emb)
